# Initial kernel scaffold; baseline (speedup 1.0000x reference)
import jax
import jax.numpy as jnp
from jax.experimental import pallas as pl


def kernel(output, ind, target, radius):
    raise NotImplementedError("write your pallas kernel here")


# speedup vs baseline: 192.7351x; 192.7351x over previous
import jax
import jax.numpy as jnp
from jax import lax
from jax.experimental import pallas as pl
from jax.experimental.pallas import tpu as pltpu
from jax.experimental.pallas import tpu_sc as plsc

DIMS = 4
W = 72
NPIX = W * W
NWORK = 27
CHUNK = NPIX // NWORK
NVEC = CHUNK // 16
RWIN = 10


def _iou_body(feat_hbm, hw_hbm, ints_hbm, tgt_hbm, out_hbm,
              feat_v, hw_v, out_v, ints_v, tgt_v):
    wid = lax.axis_index("s") * 2 + lax.axis_index("c")

    @pl.when(wid < NWORK)
    def _():
        base = wid * CHUNK
        pltpu.sync_copy(ints_hbm, ints_v)
        pltpu.sync_copy(tgt_hbm, tgt_v)
        pltpu.sync_copy(hw_hbm.at[pl.ds(base, CHUNK)], hw_v.at[pl.ds(0, CHUNK)])
        pltpu.sync_copy(hw_hbm.at[pl.ds(NPIX + base, CHUNK)],
                        hw_v.at[pl.ds(CHUNK, CHUNK)])
        for c in range(DIMS):
            pltpu.sync_copy(
                feat_hbm.at[pl.ds(c * NPIX + base, CHUNK)],
                feat_v.at[pl.ds(c * CHUNK, CHUNK)],
            )
        chv = ints_v[pl.ds(0, 16)]
        cwv = ints_v[pl.ds(16, 16)]
        rad = ints_v[pl.ds(32, 16)]
        tl = tgt_v[pl.ds(0, 16)]
        tr = tgt_v[pl.ds(16, 16)]
        tt = tgt_v[pl.ds(32, 16)]
        tb = tgt_v[pl.ds(48, 16)]
        for j in range(NVEC):
            h = hw_v[pl.ds(j * 16, 16)]
            w = hw_v[pl.ds(CHUNK + j * 16, 16)]
            rh = h - chv
            rw = w - cwv
            rha = jnp.abs(rh)
            rwa = jnp.abs(rw)
            inwin = (rha <= RWIN) & (rwa <= RWIN) & (rha <= rad) & (rwa <= rad)
            rhf = rh.astype(jnp.float32)
            rwf = rw.astype(jnp.float32)
            twl = tl + rwf
            twr = tr - rwf
            tht = tt + rhf
            thb = tb - rhf
            okt = (twl >= 0.0) & (twr >= 0.0) & (tht >= 0.0) & (thb >= 0.0)
            p0 = feat_v[pl.ds(0 * CHUNK + j * 16, 16)]
            p1 = feat_v[pl.ds(1 * CHUNK + j * 16, 16)]
            p2 = feat_v[pl.ds(2 * CHUNK + j * 16, 16)]
            p3 = feat_v[pl.ds(3 * CHUNK + j * 16, 16)]
            t_area = (twl + twr) * (tht + thb)
            p_area = (p0 + p1) * (p2 + p3)
            w_int = jnp.minimum(p0, twl) + jnp.minimum(p1, twr)
            h_int = jnp.minimum(p3, thb) + jnp.minimum(p2, tht)
            a_int = w_int * h_int
            a_un = t_area + p_area - a_int
            iou = (a_int + 1.0) / (a_un + 1.0)
            out_v[pl.ds(j * 16, 16)] = jnp.where(inwin & okt, iou, 0.0)
        pltpu.sync_copy(out_v, out_hbm.at[pl.ds(base, CHUNK)])


def kernel(output, ind, target, radius):
    feat = output.reshape(-1, DIMS, NPIX)[0].reshape(DIMS * NPIX)
    pix = jnp.arange(NPIX, dtype=jnp.int32)
    hw = jnp.concatenate([pix // W, pix % W])
    cen = ind.reshape(-1)[0].astype(jnp.int32)
    ints = jnp.concatenate([
        jnp.broadcast_to(cen // W, (16,)),
        jnp.broadcast_to(cen % W, (16,)),
        jnp.broadcast_to(jnp.asarray(radius, jnp.int32), (16,)),
    ])
    tgt = jnp.broadcast_to(
        target.reshape(-1, DIMS)[0][:, None], (DIMS, 16)
    ).reshape(DIMS * 16)
    mesh = plsc.VectorSubcoreMesh(core_axis_name="c", subcore_axis_name="s")
    iou_flat = pl.kernel(
        _iou_body,
        mesh=mesh,
        out_type=jax.ShapeDtypeStruct((NPIX,), jnp.float32),
        scratch_types=[
            pltpu.VMEM((DIMS * CHUNK,), jnp.float32),
            pltpu.VMEM((2 * CHUNK,), jnp.int32),
            pltpu.VMEM((CHUNK,), jnp.float32),
            pltpu.VMEM((48,), jnp.int32),
            pltpu.VMEM((DIMS * 16,), jnp.float32),
        ],
    )(feat, hw, ints, tgt)
    return iou_flat.reshape(W, W)
